# Initial kernel scaffold; baseline (speedup 1.0000x reference)
import jax
import jax.numpy as jnp
from jax.experimental import pallas as pl


def kernel(x, prev_attn_map, W_qkv, b_qkv, W_proj, b_proj, lepe_kernel, lepe_bias):
    raise NotImplementedError("write your pallas kernel here")


# speedup vs baseline: 3.3481x; 3.3481x over previous
import jax
import jax.numpy as jnp
from jax.experimental import pallas as pl
from jax.experimental.pallas import tpu as pltpu

_DIM = 384
_HEADS = 6
_HD = _DIM // _HEADS
_SCALE = _HD ** -0.5
_N = 1024
_SH = 32


def _fused_kernel(x_ref, prev_ref, wqkv_ref, bqkv_ref, wproj_ref, bproj_ref,
                  lk_ref, lb_ref, attn_ref, out_ref):
    h = pl.program_id(1)
    xb = x_ref[0]
    qkv = jnp.dot(xb, wqkv_ref[0], preferred_element_type=jnp.float32) + bqkv_ref[0, 0]
    q = qkv[:, :_HD]
    k = qkv[:, _HD:2 * _HD]
    v = qkv[:, 2 * _HD:]

    s = jax.lax.dot_general(q, k, (((1,), (1,)), ((), ())),
                            preferred_element_type=jnp.float32)
    s = (s * _SCALE) * prev_ref[0, 0]
    m = jnp.max(s, axis=-1, keepdims=True)
    e = jnp.exp(s - m)
    a = e * (1.0 / jnp.sum(e, axis=-1, keepdims=True))
    attn_ref[0, 0] = a
    o = jnp.dot(a, v, preferred_element_type=jnp.float32)

    vs = v.reshape(_SH, _SH, _HD)
    vp = jnp.pad(vs, ((1, 1), (1, 1), (0, 0)))
    lk = lk_ref[0]
    lep = lb_ref[0, 0] * jnp.ones((_SH, _SH, _HD), jnp.float32)
    for di in range(3):
        for dj in range(3):
            lep = lep + vp[di:di + _SH, dj:dj + _SH, :] * lk[di * 3 + dj]
    o = o + lep.reshape(_N, _HD)

    part = jnp.dot(o, wproj_ref[0], preferred_element_type=jnp.float32)

    @pl.when(h == 0)
    def _():
        out_ref[0] = part + bproj_ref[0]

    @pl.when(h != 0)
    def _():
        out_ref[0] = out_ref[0] + part


def kernel(x, prev_attn_map, W_qkv, b_qkv, W_proj, b_proj, lepe_kernel, lepe_bias):
    Bs, Hh, Ww, C = x.shape
    xf = x.reshape(Bs, _N, _DIM)
    wqkv_h = W_qkv.reshape(_DIM, 3, _HEADS, _HD).transpose(2, 0, 1, 3).reshape(_HEADS, _DIM, 3 * _HD)
    bqkv_h = b_qkv.reshape(3, _HEADS, _HD).transpose(1, 0, 2).reshape(_HEADS, 1, 3 * _HD)
    wproj_h = W_proj.reshape(_HEADS, _HD, _DIM)
    bproj = b_proj.reshape(1, _DIM)
    lk_h = lepe_kernel.reshape(9, _HEADS, _HD).transpose(1, 0, 2)
    lb_h = lepe_bias.reshape(_HEADS, 1, _HD)

    attn, out_flat = pl.pallas_call(
        _fused_kernel,
        grid=(Bs, _HEADS),
        in_specs=[
            pl.BlockSpec((1, _N, _DIM), lambda b, h: (b, 0, 0)),
            pl.BlockSpec((1, 1, _N, _N), lambda b, h: (b, h, 0, 0)),
            pl.BlockSpec((1, _DIM, 3 * _HD), lambda b, h: (h, 0, 0)),
            pl.BlockSpec((1, 1, 3 * _HD), lambda b, h: (h, 0, 0)),
            pl.BlockSpec((1, _HD, _DIM), lambda b, h: (h, 0, 0)),
            pl.BlockSpec((1, _DIM), lambda b, h: (0, 0)),
            pl.BlockSpec((1, 9, _HD), lambda b, h: (h, 0, 0)),
            pl.BlockSpec((1, 1, _HD), lambda b, h: (h, 0, 0)),
        ],
        out_specs=[
            pl.BlockSpec((1, 1, _N, _N), lambda b, h: (b, h, 0, 0)),
            pl.BlockSpec((1, _N, _DIM), lambda b, h: (b, 0, 0)),
        ],
        out_shape=[
            jax.ShapeDtypeStruct((Bs, _HEADS, _N, _N), jnp.float32),
            jax.ShapeDtypeStruct((Bs, _N, _DIM), jnp.float32),
        ],
        compiler_params=pltpu.CompilerParams(
            dimension_semantics=("parallel", "arbitrary"),
        ),
    )(xf, prev_attn_map, wqkv_h, bqkv_h, wproj_h, bproj, lk_h, lb_h)

    return out_flat.reshape(Bs, Hh, Ww, C), attn
